# baseline (speedup 1.0000x reference)
import functools

import jax
import jax.numpy as jnp
from jax import lax
from jax.experimental import pallas as pl
from jax.experimental.pallas import tpu as pltpu
from jax.experimental.pallas import tpu_sc as plsc

N = 10000
D_IN = 128
D_HID = 256
E = 320000
N_NEG = 5

_NC, _NS, _L = 2, 16, 16
_NW = _NC * _NS
_EPT = E // _NW

_sc_mesh = plsc.VectorSubcoreMesh(core_axis_name="c", subcore_axis_name="s")


@functools.partial(
    pl.kernel,
    out_type=jax.ShapeDtypeStruct((_NW, N), jnp.int32),
    mesh=_sc_mesh,
    scratch_types=[
        pltpu.VMEM((_EPT,), jnp.int32),
        pltpu.VMEM((N,), jnp.int32),
    ],
    compiler_params=pltpu.CompilerParams(needs_layout_passes=False),
)
def _deg_kernel(dst_hbm, out_hbm, idx_v, hist_v):
    wid = lax.axis_index("s") * _NC + lax.axis_index("c")
    base = wid * _EPT
    pltpu.sync_copy(dst_hbm.at[pl.ds(base, _EPT)], idx_v)
    zeros = jnp.zeros((_L,), jnp.int32)

    def zbody(i, _):
        hist_v[pl.ds(i * _L, _L)] = zeros
        return ()

    lax.fori_loop(0, N // _L, zbody, (), unroll=8)

    def body(i, _):
        d = idx_v[pl.ds(i * _L, _L)]
        cnt, last = plsc.scan_count(d)
        plsc.addupdate_scatter(hist_v, [d], cnt, mask=last)
        return ()

    lax.fori_loop(0, _EPT // _L, body, (), unroll=8)
    pltpu.sync_copy(hist_v, out_hbm.at[wid])


_ACPT = 160
_NCHUNK = _NS * _ACPT
_EPAD = _NCHUNK * 128
_STRIPE = 640
_RB = 80


@functools.partial(
    pl.kernel,
    out_type=jax.ShapeDtypeStruct((2 * N, 128), jnp.float32),
    mesh=_sc_mesh,
    scratch_types=[
        pltpu.VMEM((4, 128), jnp.int32),
        pltpu.VMEM((4, 128), jnp.int32),
        pltpu.VMEM((128, 128), jnp.float32),
        pltpu.VMEM((128, 128), jnp.float32),
        pltpu.VMEM_SHARED((N, 128), jnp.float32),
        pltpu.SemaphoreType.DMA,
        pltpu.SemaphoreType.DMA,
        pltpu.SemaphoreType.DMA,
        pltpu.SemaphoreType.DMA,
    ],
    compiler_params=pltpu.CompilerParams(needs_layout_passes=False),
)
def _agg_kernel(src_hbm, dst_hbm, y_hbm, out_hbm, sidx_v, didx_v, rows0_v,
                rows1_v, acc_sh, gsem0, gsem1, isem0, isem1):
    c = lax.axis_index("c")
    sid = lax.axis_index("s")
    zeros = jnp.zeros((_L,), jnp.float32)
    cbase = sid * _ACPT
    coff = jnp.broadcast_to(c * N, (_L,)).astype(jnp.int32)
    zrow = jnp.broadcast_to(2 * N, (_L,)).astype(jnp.int32)

    def adj(r):
        for k in range(128 // _L):
            sl = pl.ds(k * _L, _L)
            sidx_v[r, sl] = jnp.minimum(sidx_v[r, sl] + coff, zrow)

    def zb(i, _):
        for k in range(128 // _L):
            rows0_v[i, pl.ds(k * _L, _L)] = zeros
        return ()

    lax.fori_loop(0, _RB, zb, (), unroll=4)
    nrb = jnp.where(sid < _NS - 1, _STRIPE // _RB, (N - 15 * _STRIPE) // _RB)

    def zs(k, _):
        r0 = pl.multiple_of(sid * _STRIPE + k * _RB, _RB)
        pltpu.sync_copy(rows0_v.at[pl.ds(0, _RB)], acc_sh.at[pl.ds(r0, _RB)])
        return ()

    lax.fori_loop(0, nrb, zs, ())
    plsc.subcore_barrier()

    rows = (rows0_v, rows1_v)
    gsem = (gsem0, gsem1)
    isem = (isem0, isem1)

    def outer(j, _):
        cb = pl.multiple_of((cbase + j) * 128, 128)
        pltpu.sync_copy(src_hbm.at[pl.ds(cb, 128)], sidx_v.at[0])
        pltpu.sync_copy(dst_hbm.at[pl.ds(cb, 128)], didx_v.at[0])
        adj(0)
        pltpu.async_copy(y_hbm.at[sidx_v.at[0]], rows0_v, gsem0).wait()
        pltpu.sync_copy(rows0_v, acc_sh.at[didx_v.at[0]], add=True)
        return ()

    lax.fori_loop(0, _ACPT, outer, ())
    plsc.subcore_barrier()

    def wb(k, _):
        r0 = pl.multiple_of(sid * _STRIPE + k * _RB, _RB)
        pltpu.sync_copy(acc_sh.at[pl.ds(r0, _RB)], rows0_v.at[pl.ds(0, _RB)])
        pltpu.sync_copy(rows0_v.at[pl.ds(0, _RB)],
                        out_hbm.at[pl.ds(c * N + r0, _RB)])
        return ()

    lax.fori_loop(0, nrb, wb, ())


_NPAIR = E + N_NEG * N
_NPCH = -(-_NPAIR // 128)
_NPAD = _NPCH * 128
_PCPT = _NPCH // _NW
_PCREM = _NPCH - _PCPT * _NW


@functools.partial(
    pl.kernel,
    out_type=jax.ShapeDtypeStruct((_NPAD,), jnp.float32),
    mesh=_sc_mesh,
    scratch_types=[
        pltpu.VMEM((1, 128), jnp.int32),
        pltpu.VMEM((1, 128), jnp.int32),
        pltpu.VMEM((128, D_HID), jnp.float32),
        pltpu.VMEM((128, D_HID), jnp.float32),
        pltpu.VMEM((1, 128), jnp.float32),
        pltpu.VMEM((128, _L), jnp.float32),
        pltpu.SemaphoreType.DMA,
        pltpu.SemaphoreType.DMA,
    ],
    compiler_params=pltpu.CompilerParams(needs_layout_passes=False),
)
def _sims_kernel(aidx_hbm, bidx_hbm, z_hbm, out_hbm, aidx_v, bidx_v,
                 za_v, zb_v, sims_v, accs_v, sema, semb):
    wid = lax.axis_index("s") * _NC + lax.axis_index("c")
    base = wid * _PCPT + jnp.minimum(wid, _PCREM)
    nch = _PCPT + jnp.where(wid < _PCREM, 1, 0)

    def body(j, _):
        cb = pl.multiple_of((base + j) * 128, 128)
        pltpu.sync_copy(aidx_hbm.at[pl.ds(cb, 128)], aidx_v.at[0])
        pltpu.sync_copy(bidx_hbm.at[pl.ds(cb, 128)], bidx_v.at[0])
        cpa = pltpu.async_copy(z_hbm.at[aidx_v.at[0]], za_v, sema)
        cpb = pltpu.async_copy(z_hbm.at[bidx_v.at[0]], zb_v, semb)
        cpa.wait()
        cpb.wait()

        def dot(e, _):
            acc = za_v[e, pl.ds(0, _L)] * zb_v[e, pl.ds(0, _L)]
            for k in range(1, D_HID // _L):
                sl = pl.ds(k * _L, _L)
                acc = acc + za_v[e, sl] * zb_v[e, sl]
            accs_v[e, pl.ds(0, _L)] = acc
            return ()

        lax.fori_loop(0, 128, dot, (), unroll=2)

        lanes = lax.iota(jnp.int32, _L)
        for g in range(128 // _L):
            rows = lanes + g * _L
            s = plsc.load_gather(accs_v, [rows, jnp.zeros((_L,), jnp.int32)])
            for k in range(1, _L):
                s = s + plsc.load_gather(
                    accs_v, [rows, jnp.full((_L,), k, jnp.int32)])
            sims_v[0, pl.ds(g * _L, _L)] = s
        pltpu.sync_copy(sims_v.at[0], out_hbm.at[pl.ds(cb, 128)])
        return ()

    lax.fori_loop(0, nch, body, ())


def _matmul_body(x_ref, w_ref, o_ref):
    o_ref[...] = jnp.dot(x_ref[...], w_ref[...],
                         preferred_element_type=jnp.float32)


def _matmul(x, W):
    blk = 1000
    return pl.pallas_call(
        _matmul_body,
        grid=(N // blk,),
        in_specs=[
            pl.BlockSpec((blk, D_IN), lambda i: (i, 0)),
            pl.BlockSpec((D_IN, D_HID), lambda i: (0, 0)),
        ],
        out_specs=pl.BlockSpec((blk, D_HID), lambda i: (i, 0)),
        out_shape=jax.ShapeDtypeStruct((N, D_HID), jnp.float32),
    )(x, W)


def kernel(node_features, edge_indices, W, b):
    x = node_features
    src = edge_indices[0]
    dst = edge_indices[1]
    deg_part = _deg_kernel(dst)
    deg = deg_part.sum(axis=0).astype(jnp.float32) + 1.0
    dinv = jax.lax.rsqrt(deg)
    xw = _matmul(x, W)
    y = xw * dinv[:, None]
    y_flat = jnp.concatenate(
        [y[:, :128], y[:, 128:], jnp.zeros((8, 128), jnp.float32)], axis=0)
    epad = jnp.zeros((_EPAD - E,), jnp.int32)
    src_pad = jnp.concatenate([src, epad + 2 * N])
    dst_pad = jnp.concatenate([dst, epad])
    acc_flat = _agg_kernel(src_pad, dst_pad, y_flat)
    acc = jnp.concatenate([acc_flat[:N], acc_flat[N:]], axis=1)
    out = dinv[:, None] * (acc + y) + b
    h = jax.nn.relu(out)
    nrm = jnp.linalg.norm(h, axis=1, keepdims=True)
    z = h / jnp.maximum(nrm, 1e-12)

    nk = jax.random.key(12345)
    neg = jax.random.randint(nk, (2, N_NEG * N), 0, N, dtype=jnp.int32)
    pad = jnp.zeros((_NPAD - _NPAIR,), jnp.int32)
    a_idx = jnp.concatenate([src, neg[0], pad])
    b_idx = jnp.concatenate([dst, neg[1], pad])
    sims = _sims_kernel(a_idx, b_idx, z)

    pos_mask = src < dst
    all_pos_sim = sims[:E]
    num_pos = jnp.sum(pos_mask)
    pos_loss = jnp.sum(jnp.where(pos_mask, (all_pos_sim - 1.0) ** 2, 0.0)) / num_pos.astype(z.dtype)
    num_neg = jnp.minimum(num_pos, N_NEG * N)
    neg_mask = (neg[0] < neg[1]) & (jnp.arange(N_NEG * N) < num_neg)
    neg_sim = sims[E:_NPAIR]
    cnt_neg = jnp.sum(neg_mask)
    neg_loss = jnp.sum(jnp.where(neg_mask, neg_sim ** 2, 0.0)) / cnt_neg.astype(z.dtype)
    return (z, pos_loss + neg_loss)


# speedup vs baseline: 1.3507x; 1.3507x over previous
import functools

import jax
import jax.numpy as jnp
from jax import lax
from jax.experimental import pallas as pl
from jax.experimental.pallas import tpu as pltpu
from jax.experimental.pallas import tpu_sc as plsc

N = 10000
D_IN = 128
D_HID = 256
E = 320000
N_NEG = 5

_NC, _NS, _L = 2, 16, 16
_NW = _NC * _NS
_EPT = E // _NW

_sc_mesh = plsc.VectorSubcoreMesh(core_axis_name="c", subcore_axis_name="s")


@functools.partial(
    pl.kernel,
    out_type=jax.ShapeDtypeStruct((_NW, N), jnp.int32),
    mesh=_sc_mesh,
    scratch_types=[
        pltpu.VMEM((_EPT,), jnp.int32),
        pltpu.VMEM((N,), jnp.int32),
    ],
    compiler_params=pltpu.CompilerParams(needs_layout_passes=False),
)
def _deg_kernel(dst_hbm, out_hbm, idx_v, hist_v):
    wid = lax.axis_index("s") * _NC + lax.axis_index("c")
    base = wid * _EPT
    pltpu.sync_copy(dst_hbm.at[pl.ds(base, _EPT)], idx_v)
    zeros = jnp.zeros((_L,), jnp.int32)

    def zbody(i, _):
        hist_v[pl.ds(i * _L, _L)] = zeros
        return ()

    lax.fori_loop(0, N // _L, zbody, (), unroll=8)

    def body(i, _):
        d = idx_v[pl.ds(i * _L, _L)]
        cnt, last = plsc.scan_count(d)
        plsc.addupdate_scatter(hist_v, [d], cnt, mask=last)
        return ()

    lax.fori_loop(0, _EPT // _L, body, (), unroll=8)
    pltpu.sync_copy(hist_v, out_hbm.at[wid])


_NCHUNK = E // 128
_CPT = _NCHUNK // _NS
_CREM = _NCHUNK - _CPT * _NS
_STRIPE = 640
_RB = 80


@functools.partial(
    pl.kernel,
    out_type=jax.ShapeDtypeStruct((2 * N, 128), jnp.float32),
    mesh=_sc_mesh,
    scratch_types=[
        pltpu.VMEM((1, 128), jnp.int32),
        pltpu.VMEM((1, 128), jnp.int32),
        pltpu.VMEM((128, 128), jnp.float32),
        pltpu.VMEM((_RB, 128), jnp.float32),
        pltpu.VMEM_SHARED((N, 128), jnp.float32),
        pltpu.SemaphoreType.DMA,
    ],
    compiler_params=pltpu.CompilerParams(needs_layout_passes=False),
)
def _agg_kernel(src_hbm, dst_hbm, y_hbm, out_hbm, sidx_v, didx_v, rows_v,
                buf_v, acc_sh, sem):
    c = lax.axis_index("c")
    sid = lax.axis_index("s")
    zeros = jnp.zeros((_L,), jnp.float32)

    def zb(i, _):
        for k in range(128 // _L):
            buf_v[i, pl.ds(k * _L, _L)] = zeros
        return ()

    lax.fori_loop(0, _RB, zb, (), unroll=4)
    nrb = jnp.where(sid < _NS - 1, _STRIPE // _RB, (N - 15 * _STRIPE) // _RB)

    def zs(k, _):
        r0 = pl.multiple_of(sid * _STRIPE + k * _RB, _RB)
        pltpu.sync_copy(buf_v, acc_sh.at[pl.ds(r0, _RB)])
        return ()

    lax.fori_loop(0, nrb, zs, ())
    plsc.subcore_barrier()

    base = sid * _CPT + jnp.minimum(sid, _CREM)
    nch = _CPT + jnp.where(sid < _CREM, 1, 0)
    coff = jnp.broadcast_to(c * N, (_L,)).astype(jnp.int32)

    def body(j, _):
        cb = pl.multiple_of((base + j) * 128, 128)
        pltpu.sync_copy(src_hbm.at[pl.ds(cb, 128)], sidx_v.at[0])
        pltpu.sync_copy(dst_hbm.at[pl.ds(cb, 128)], didx_v.at[0])
        for k in range(128 // _L):
            sl = pl.ds(k * _L, _L)
            sidx_v[0, sl] = sidx_v[0, sl] + coff
        pltpu.async_copy(y_hbm.at[sidx_v.at[0]], rows_v, sem).wait()
        pltpu.sync_copy(rows_v, acc_sh.at[didx_v.at[0]], add=True)
        return ()

    lax.fori_loop(0, nch, body, ())
    plsc.subcore_barrier()

    def wb(k, _):
        r0 = pl.multiple_of(sid * _STRIPE + k * _RB, _RB)
        pltpu.sync_copy(acc_sh.at[pl.ds(r0, _RB)], buf_v)
        pltpu.sync_copy(buf_v, out_hbm.at[pl.ds(c * N + r0, _RB)])
        return ()

    lax.fori_loop(0, nrb, wb, ())


_NPAIR = E + N_NEG * N
_NPCH = -(-_NPAIR // 128)
_NPAD = _NPCH * 128
_PCPT = _NPCH // _NW
_PCREM = _NPCH - _PCPT * _NW


@functools.partial(
    pl.kernel,
    out_type=jax.ShapeDtypeStruct((_NPAD,), jnp.float32),
    mesh=_sc_mesh,
    scratch_types=[
        pltpu.VMEM((1, 128), jnp.int32),
        pltpu.VMEM((1, 128), jnp.int32),
        pltpu.VMEM((128, D_HID), jnp.float32),
        pltpu.VMEM((128, D_HID), jnp.float32),
        pltpu.VMEM((1, 128), jnp.float32),
        pltpu.VMEM((128, _L), jnp.float32),
        pltpu.SemaphoreType.DMA,
        pltpu.SemaphoreType.DMA,
    ],
    compiler_params=pltpu.CompilerParams(needs_layout_passes=False),
)
def _sims_kernel(aidx_hbm, bidx_hbm, z_hbm, out_hbm, aidx_v, bidx_v,
                 za_v, zb_v, sims_v, accs_v, sema, semb):
    wid = lax.axis_index("s") * _NC + lax.axis_index("c")
    base = wid * _PCPT + jnp.minimum(wid, _PCREM)
    nch = _PCPT + jnp.where(wid < _PCREM, 1, 0)

    def body(j, _):
        cb = pl.multiple_of((base + j) * 128, 128)
        pltpu.sync_copy(aidx_hbm.at[pl.ds(cb, 128)], aidx_v.at[0])
        pltpu.sync_copy(bidx_hbm.at[pl.ds(cb, 128)], bidx_v.at[0])
        cpa = pltpu.async_copy(z_hbm.at[aidx_v.at[0]], za_v, sema)
        cpb = pltpu.async_copy(z_hbm.at[bidx_v.at[0]], zb_v, semb)
        cpa.wait()
        cpb.wait()

        def dot(e, _):
            acc = za_v[e, pl.ds(0, _L)] * zb_v[e, pl.ds(0, _L)]
            for k in range(1, D_HID // _L):
                sl = pl.ds(k * _L, _L)
                acc = acc + za_v[e, sl] * zb_v[e, sl]
            accs_v[e, pl.ds(0, _L)] = acc
            return ()

        lax.fori_loop(0, 128, dot, (), unroll=2)

        lanes = lax.iota(jnp.int32, _L)
        for g in range(128 // _L):
            rows = lanes + g * _L
            s = plsc.load_gather(accs_v, [rows, jnp.zeros((_L,), jnp.int32)])
            for k in range(1, _L):
                s = s + plsc.load_gather(
                    accs_v, [rows, jnp.full((_L,), k, jnp.int32)])
            sims_v[0, pl.ds(g * _L, _L)] = s
        pltpu.sync_copy(sims_v.at[0], out_hbm.at[pl.ds(cb, 128)])
        return ()

    lax.fori_loop(0, nch, body, ())


def _matmul_body(x_ref, w_ref, o_ref):
    o_ref[...] = jnp.dot(x_ref[...], w_ref[...],
                         preferred_element_type=jnp.float32)


def _matmul(x, W):
    blk = 1000
    return pl.pallas_call(
        _matmul_body,
        grid=(N // blk,),
        in_specs=[
            pl.BlockSpec((blk, D_IN), lambda i: (i, 0)),
            pl.BlockSpec((D_IN, D_HID), lambda i: (0, 0)),
        ],
        out_specs=pl.BlockSpec((blk, D_HID), lambda i: (i, 0)),
        out_shape=jax.ShapeDtypeStruct((N, D_HID), jnp.float32),
    )(x, W)


def kernel(node_features, edge_indices, W, b):
    x = node_features
    src = edge_indices[0]
    dst = edge_indices[1]
    deg_part = _deg_kernel(dst)
    deg = deg_part.sum(axis=0).astype(jnp.float32) + 1.0
    dinv = jax.lax.rsqrt(deg)
    xw = _matmul(x, W)
    y = xw * dinv[:, None]
    y_flat = jnp.concatenate([y[:, :128], y[:, 128:]], axis=0)
    acc_flat = _agg_kernel(src, dst, y_flat)
    acc = jnp.concatenate([acc_flat[:N], acc_flat[N:]], axis=1)
    out = dinv[:, None] * (acc + y) + b
    h = jax.nn.relu(out)
    nrm = jnp.linalg.norm(h, axis=1, keepdims=True)
    z = h / jnp.maximum(nrm, 1e-12)

    nk = jax.random.key(12345)
    neg = jax.random.randint(nk, (2, N_NEG * N), 0, N, dtype=jnp.int32)
    pad = jnp.zeros((_NPAD - _NPAIR,), jnp.int32)
    a_idx = jnp.concatenate([src, neg[0], pad])
    b_idx = jnp.concatenate([dst, neg[1], pad])
    sims = _sims_kernel(a_idx, b_idx, z)

    pos_mask = src < dst
    all_pos_sim = sims[:E]
    num_pos = jnp.sum(pos_mask)
    pos_loss = jnp.sum(jnp.where(pos_mask, (all_pos_sim - 1.0) ** 2, 0.0)) / num_pos.astype(z.dtype)
    num_neg = jnp.minimum(num_pos, N_NEG * N)
    neg_mask = (neg[0] < neg[1]) & (jnp.arange(N_NEG * N) < num_neg)
    neg_sim = sims[E:_NPAIR]
    cnt_neg = jnp.sum(neg_mask)
    neg_loss = jnp.sum(jnp.where(neg_mask, neg_sim ** 2, 0.0)) / cnt_neg.astype(z.dtype)
    return (z, pos_loss + neg_loss)


# speedup vs baseline: 1.4832x; 1.0980x over previous
import functools

import jax
import jax.numpy as jnp
from jax import lax
from jax.experimental import pallas as pl
from jax.experimental.pallas import tpu as pltpu
from jax.experimental.pallas import tpu_sc as plsc

N = 10000
D_IN = 128
D_HID = 256
E = 320000
N_NEG = 5

_NC, _NS, _L = 2, 16, 16
_NW = _NC * _NS
_EPT = E // _NW

_sc_mesh = plsc.VectorSubcoreMesh(core_axis_name="c", subcore_axis_name="s")


@functools.partial(
    pl.kernel,
    out_type=jax.ShapeDtypeStruct((_NW, N), jnp.int32),
    mesh=_sc_mesh,
    scratch_types=[
        pltpu.VMEM((_EPT,), jnp.int32),
        pltpu.VMEM((N,), jnp.int32),
    ],
    compiler_params=pltpu.CompilerParams(needs_layout_passes=False),
)
def _deg_kernel(dst_hbm, out_hbm, idx_v, hist_v):
    wid = lax.axis_index("s") * _NC + lax.axis_index("c")
    base = wid * _EPT
    pltpu.sync_copy(dst_hbm.at[pl.ds(base, _EPT)], idx_v)
    zeros = jnp.zeros((_L,), jnp.int32)

    def zbody(i, _):
        hist_v[pl.ds(i * _L, _L)] = zeros
        return ()

    lax.fori_loop(0, N // _L, zbody, (), unroll=8)

    def body(i, _):
        d = idx_v[pl.ds(i * _L, _L)]
        cnt, last = plsc.scan_count(d)
        plsc.addupdate_scatter(hist_v, [d], cnt, mask=last)
        return ()

    lax.fori_loop(0, _EPT // _L, body, (), unroll=8)
    pltpu.sync_copy(hist_v, out_hbm.at[wid])


_NCHUNK = E // 128
_CPT = _NCHUNK // _NS
_CREM = _NCHUNK - _CPT * _NS
_STRIPE = 640
_RB = 80


@functools.partial(
    pl.kernel,
    out_type=jax.ShapeDtypeStruct((2 * N, 128), jnp.float32),
    mesh=_sc_mesh,
    scratch_types=[
        pltpu.VMEM((1, 128), jnp.int32),
        pltpu.VMEM((1, 128), jnp.int32),
        pltpu.VMEM((128, 128), jnp.float32),
        pltpu.VMEM((_RB, 128), jnp.float32),
        pltpu.VMEM_SHARED((N, 128), jnp.float32),
        pltpu.SemaphoreType.DMA,
    ],
    compiler_params=pltpu.CompilerParams(needs_layout_passes=False),
)
def _agg_kernel(src_hbm, dst_hbm, y_hbm, out_hbm, sidx_v, didx_v, rows_v,
                buf_v, acc_sh, sem):
    c = lax.axis_index("c")
    sid = lax.axis_index("s")
    zeros = jnp.zeros((_L,), jnp.float32)

    def zb(i, _):
        for k in range(128 // _L):
            buf_v[i, pl.ds(k * _L, _L)] = zeros
        return ()

    lax.fori_loop(0, _RB, zb, (), unroll=4)
    nrb = jnp.where(sid < _NS - 1, _STRIPE // _RB, (N - 15 * _STRIPE) // _RB)

    def zs(k, _):
        r0 = pl.multiple_of(sid * _STRIPE + k * _RB, _RB)
        pltpu.sync_copy(buf_v, acc_sh.at[pl.ds(r0, _RB)])
        return ()

    lax.fori_loop(0, nrb, zs, ())
    plsc.subcore_barrier()

    base = sid * _CPT + jnp.minimum(sid, _CREM)
    nch = _CPT + jnp.where(sid < _CREM, 1, 0)
    coff = jnp.broadcast_to(c * N, (_L,)).astype(jnp.int32)

    def body(j, _):
        cb = pl.multiple_of((base + j) * 128, 128)
        pltpu.sync_copy(src_hbm.at[pl.ds(cb, 128)], sidx_v.at[0])
        pltpu.sync_copy(dst_hbm.at[pl.ds(cb, 128)], didx_v.at[0])
        for k in range(128 // _L):
            sl = pl.ds(k * _L, _L)
            sidx_v[0, sl] = sidx_v[0, sl] + coff
        pltpu.async_copy(y_hbm.at[sidx_v.at[0]], rows_v, sem).wait()
        pltpu.sync_copy(rows_v, acc_sh.at[didx_v.at[0]], add=True)
        return ()

    lax.fori_loop(0, nch, body, ())
    plsc.subcore_barrier()

    def wb(k, _):
        r0 = pl.multiple_of(sid * _STRIPE + k * _RB, _RB)
        pltpu.sync_copy(acc_sh.at[pl.ds(r0, _RB)], buf_v)
        pltpu.sync_copy(buf_v, out_hbm.at[pl.ds(c * N + r0, _RB)])
        return ()

    lax.fori_loop(0, nrb, wb, ())


_NPAIR = E + N_NEG * N
_NPCH = -(-_NPAIR // 128)
_NPAD = _NPCH * 128
_PCPT = _NPCH // _NW
_PCREM = _NPCH - _PCPT * _NW


@functools.partial(
    pl.kernel,
    out_type=jax.ShapeDtypeStruct((_NPAD,), jnp.float32),
    mesh=_sc_mesh,
    scratch_types=[
        pltpu.VMEM((1, 128), jnp.int32),
        pltpu.VMEM((1, 128), jnp.int32),
        pltpu.VMEM((128, 128), jnp.int32),
        pltpu.VMEM((128, 128), jnp.int32),
        pltpu.VMEM((1, 128), jnp.float32),
        pltpu.VMEM((128, _L), jnp.float32),
        pltpu.SemaphoreType.DMA,
        pltpu.SemaphoreType.DMA,
    ],
    compiler_params=pltpu.CompilerParams(needs_layout_passes=False),
)
def _sims_kernel(aidx_hbm, bidx_hbm, z_hbm, out_hbm, aidx_v, bidx_v,
                 za_v, zb_v, sims_v, accs_v, sema, semb):
    wid = lax.axis_index("s") * _NC + lax.axis_index("c")
    base = wid * _PCPT + jnp.minimum(wid, _PCREM)
    nch = _PCPT + jnp.where(wid < _PCREM, 1, 0)

    def body(j, _):
        cb = pl.multiple_of((base + j) * 128, 128)
        pltpu.sync_copy(aidx_hbm.at[pl.ds(cb, 128)], aidx_v.at[0])
        pltpu.sync_copy(bidx_hbm.at[pl.ds(cb, 128)], bidx_v.at[0])
        cpa = pltpu.async_copy(z_hbm.at[aidx_v.at[0]], za_v, sema)
        cpb = pltpu.async_copy(z_hbm.at[bidx_v.at[0]], zb_v, semb)
        cpa.wait()
        cpb.wait()

        def dot(e, _):
            acc = None
            for k in range(128 // _L):
                sl = pl.ds(k * _L, _L)
                a = plsc.bitcast(za_v[e, sl], jnp.bfloat16)
                bb = plsc.bitcast(zb_v[e, sl], jnp.bfloat16)
                acc = a * bb if acc is None else acc + a * bb
            lo, hi = plsc.unpack(acc, format=plsc.PackFormat.INTERLEAVED)
            accs_v[e, pl.ds(0, _L)] = lo + hi
            return ()

        lax.fori_loop(0, 128, dot, (), unroll=2)

        lanes = lax.iota(jnp.int32, _L)
        for g in range(128 // _L):
            rows = lanes + g * _L
            s = plsc.load_gather(accs_v, [rows, jnp.zeros((_L,), jnp.int32)])
            for k in range(1, _L):
                s = s + plsc.load_gather(
                    accs_v, [rows, jnp.full((_L,), k, jnp.int32)])
            sims_v[0, pl.ds(g * _L, _L)] = s
        pltpu.sync_copy(sims_v.at[0], out_hbm.at[pl.ds(cb, 128)])
        return ()

    lax.fori_loop(0, nch, body, ())


def _matmul_body(x_ref, w_ref, o_ref):
    o_ref[...] = jnp.dot(x_ref[...], w_ref[...],
                         preferred_element_type=jnp.float32)


def _matmul(x, W):
    blk = 1000
    return pl.pallas_call(
        _matmul_body,
        grid=(N // blk,),
        in_specs=[
            pl.BlockSpec((blk, D_IN), lambda i: (i, 0)),
            pl.BlockSpec((D_IN, D_HID), lambda i: (0, 0)),
        ],
        out_specs=pl.BlockSpec((blk, D_HID), lambda i: (i, 0)),
        out_shape=jax.ShapeDtypeStruct((N, D_HID), jnp.float32),
    )(x, W)


def kernel(node_features, edge_indices, W, b):
    x = node_features
    src = edge_indices[0]
    dst = edge_indices[1]
    deg_part = _deg_kernel(dst)
    deg = deg_part.sum(axis=0).astype(jnp.float32) + 1.0
    dinv = jax.lax.rsqrt(deg)
    xw = _matmul(x, W)
    y = xw * dinv[:, None]
    y_flat = jnp.concatenate([y[:, :128], y[:, 128:]], axis=0)
    acc_flat = _agg_kernel(src, dst, y_flat)
    acc = jnp.concatenate([acc_flat[:N], acc_flat[N:]], axis=1)
    out = dinv[:, None] * (acc + y) + b
    h = jax.nn.relu(out)
    nrm = jnp.linalg.norm(h, axis=1, keepdims=True)
    z = h / jnp.maximum(nrm, 1e-12)

    nk = jax.random.key(12345)
    neg = jax.random.randint(nk, (2, N_NEG * N), 0, N, dtype=jnp.int32)
    pad = jnp.zeros((_NPAD - _NPAIR,), jnp.int32)
    a_idx = jnp.concatenate([src, neg[0], pad])
    b_idx = jnp.concatenate([dst, neg[1], pad])
    z_pack = jax.lax.bitcast_convert_type(
        z.astype(jnp.bfloat16).reshape(N, 128, 2), jnp.int32)
    sims = _sims_kernel(a_idx, b_idx, z_pack)

    pos_mask = src < dst
    all_pos_sim = sims[:E]
    num_pos = jnp.sum(pos_mask)
    pos_loss = jnp.sum(jnp.where(pos_mask, (all_pos_sim - 1.0) ** 2, 0.0)) / num_pos.astype(z.dtype)
    num_neg = jnp.minimum(num_pos, N_NEG * N)
    neg_mask = (neg[0] < neg[1]) & (jnp.arange(N_NEG * N) < num_neg)
    neg_sim = sims[E:_NPAIR]
    cnt_neg = jnp.sum(neg_mask)
    neg_loss = jnp.sum(jnp.where(neg_mask, neg_sim ** 2, 0.0)) / cnt_neg.astype(z.dtype)
    return (z, pos_loss + neg_loss)
